# baseline (speedup 1.0000x reference)
import functools

import jax
import jax.numpy as jnp
import numpy as np
from jax import lax
from jax.experimental import pallas as pl
from jax.experimental.pallas import tpu as pltpu
from jax.experimental.pallas import tpu_sc as plsc

N = 10000
D = 128
E = 320000
ED = 7

NC = 2
NS = 16
NW = NC * NS
CH = 128
CPS = 4
SUP_E = CPS * CH
NSUP = 20
EPT = NSUP * SUP_E
E_PAD = EPT * NW
ROWS_PT = 624
TAIL = N - NS * ROWS_PT
PHASES = 4
NCH = D // 16


_RB = 1000


def _pre_body(x_ref, w_ref, wr_ref, bias_ref, degs_ref, perm_ref,
              table_ref, self_ref, res_ref):
    x = x_ref[...]
    nf = jnp.dot(x, w_ref[...], preferred_element_type=jnp.float32) + bias_ref[0]
    table_ref[...] = nf + bias_ref[1]
    self_ref[...] = jnp.maximum(nf + bias_ref[3], 0.0) / degs_ref[...]
    res_ref[...] = jnp.maximum(
        jnp.dot(x, wr_ref[...], preferred_element_type=jnp.float32) + bias_ref[2],
        0.0)


def _pre_call(x, W, Wr, bias, degs, perm):
    return pl.pallas_call(
        _pre_body,
        grid=(N // _RB,),
        in_specs=[
            pl.BlockSpec((_RB, D), lambda i: (i, 0)),
            pl.BlockSpec((D, D), lambda i: (0, 0)),
            pl.BlockSpec((D, D), lambda i: (0, 0)),
            pl.BlockSpec((8, D), lambda i: (0, 0)),
            pl.BlockSpec((_RB, 1), lambda i: (i, 0)),
            pl.BlockSpec((D, D), lambda i: (0, 0)),
        ],
        out_specs=[
            pl.BlockSpec((_RB, D), lambda i: (i, 0)),
            pl.BlockSpec((_RB, D), lambda i: (i, 0)),
            pl.BlockSpec((_RB, D), lambda i: (i, 0)),
        ],
        out_shape=[
            jax.ShapeDtypeStruct((N, D), jnp.float32),
            jax.ShapeDtypeStruct((N, D), jnp.float32),
            jax.ShapeDtypeStruct((N, D), jnp.float32),
        ],
    )(x, W, Wr, bias, degs, perm)


def _mp_body(table_h, sd_h, ne_h, w_h, z_h, out_h,
             w_v, sd_v, ne_v, g0, g1, s0, s1, acc,
             gsem0, gsem1, ssem0, ssem1):
    cid = lax.axis_index("c")
    sid = lax.axis_index("s")
    wid = cid * NS + sid

    pltpu.sync_copy(w_h, w_v)

    def zero_acc():
        pltpu.sync_copy(z_h.at[pl.ds(0, ROWS_PT)],
                        acc.at[pl.ds(sid * ROWS_PT, ROWS_PT)])

        @pl.when(sid == NS - 1)
        def _zero_tail():
            pltpu.sync_copy(z_h.at[pl.ds(0, TAIL)],
                            acc.at[pl.ds(NS * ROWS_PT, TAIL)])

    wv = [[w_v[k, pl.ds(c * 16, 16)] for c in range(NCH)] for k in range(ED)]

    gbufs = (g0, g1)
    sbufs = (s0, s1)
    gsems = (gsem0, gsem1)
    ssems = (ssem0, ssem1)

    def superchunk(s, carry):
        pltpu.sync_copy(sd_h.at[wid, s], sd_v)
        pltpu.sync_copy(ne_h.at[wid, s], ne_v)
        pltpu.async_copy(table_h.at[sd_v.at[0, 0]], g0, gsem0)
        for j in range(CPS):
            b = j % 2
            gbuf = gbufs[b]
            sbuf = sbufs[b]
            pltpu.make_async_copy(
                table_h.at[sd_v.at[0, j]], gbuf, gsems[b]).wait()
            if j + 1 < CPS:
                pltpu.async_copy(table_h.at[sd_v.at[0, j + 1]],
                                 gbufs[1 - b], gsems[1 - b])
            if j >= 2:
                pltpu.make_async_copy(
                    sbuf, acc.at[sd_v.at[1, j - 2]], ssems[b]).wait()

            def pair_body(i2, _, j=j, gbuf=gbuf, sbuf=sbuf):
                nev = ne_v[j * (CH // 2) + i2, :]
                for half in range(2):
                    i = 2 * i2 + half
                    f = [nev[8 * half + k] for k in range(ED)]
                    nrm = nev[8 * half + ED]
                    out = []
                    for c in range(NCH):
                        v = gbuf[i, pl.ds(c * 16, 16)]
                        for k in range(ED):
                            v = v + f[k] * wv[k][c]
                        out.append(jnp.maximum(v, 0.0) * nrm)
                    for c2 in range(NCH // 2):
                        packed = plsc.pack(out[2 * c2], out[2 * c2 + 1],
                                           format=plsc.PackFormat.INTERLEAVED)
                        sbuf[i, pl.ds(c2 * 32, 32)] = packed
                return 0

            lax.fori_loop(0, CH // 2, pair_body, 0)
            pltpu.async_copy(sbuf, acc.at[sd_v.at[1, j]], ssems[b], add=True)
        pltpu.make_async_copy(s0, acc.at[sd_v.at[1, CPS - 2]], ssem0).wait()
        pltpu.make_async_copy(s1, acc.at[sd_v.at[1, CPS - 1]], ssem1).wait()
        return carry

    for ph in range(PHASES):
        zero_acc()
        plsc.subcore_barrier()
        lax.fori_loop(ph * (NSUP // PHASES), (ph + 1) * (NSUP // PHASES),
                      superchunk, 0)
        plsc.subcore_barrier()
        o = out_h.at[NC * ph + cid]
        pltpu.sync_copy(acc.at[pl.ds(sid * ROWS_PT, ROWS_PT)],
                        o.at[pl.ds(sid * ROWS_PT, ROWS_PT)])

        @pl.when(sid == NS - 1)
        def _copy_tail(o=o):
            pltpu.sync_copy(acc.at[pl.ds(NS * ROWS_PT, TAIL)],
                            o.at[pl.ds(NS * ROWS_PT, TAIL)])
        plsc.subcore_barrier()


_mp_call = pl.kernel(
    _mp_body,
    out_type=jax.ShapeDtypeStruct((NC * PHASES, N, D), jnp.bfloat16),
    mesh=plsc.VectorSubcoreMesh(core_axis_name="c", subcore_axis_name="s"),
    compiler_params=pltpu.CompilerParams(use_tc_tiling_on_sc=False,
                                         needs_layout_passes=False),
    scratch_types=[
        pltpu.VMEM((8, D), jnp.float32),
        pltpu.VMEM((2, CPS, CH), jnp.int32),
        pltpu.VMEM((SUP_E // 2, 16), jnp.float32),
        pltpu.VMEM((CH, D), jnp.float32),
        pltpu.VMEM((CH, D), jnp.float32),
        pltpu.VMEM((CH, D), jnp.bfloat16),
        pltpu.VMEM((CH, D), jnp.bfloat16),
        pltpu.VMEM_SHARED((N, D), jnp.bfloat16),
        pltpu.SemaphoreType.DMA,
        pltpu.SemaphoreType.DMA,
        pltpu.SemaphoreType.DMA,
        pltpu.SemaphoreType.DMA,
    ],
)


def _post_body(ft2_ref, self_ref, res_ref, gb_ref, unperm_ref, y_ref):
    fts = jnp.sum(ft2_ref[...].astype(jnp.float32), axis=0)
    ft = jnp.dot(fts, unperm_ref[...], preferred_element_type=jnp.float32)
    y = jnp.maximum(ft + self_ref[...], 0.0) + res_ref[...]
    mean = jnp.mean(y, axis=0, keepdims=True)
    var = jnp.mean((y - mean) * (y - mean), axis=0, keepdims=True)
    y_ref[...] = (y - mean) / jnp.sqrt(var + 1e-5) * gb_ref[0] + gb_ref[1]


def _post_call(ft2, selfterm, res, gb, unperm):
    return pl.pallas_call(
        _post_body,
        out_shape=jax.ShapeDtypeStruct((N, D), jnp.float32),
    )(ft2, selfterm, res, gb, unperm)


def _build_unperm():
    s = np.arange(D)
    f = 32 * (s // 32) + 16 * (s % 2) + (s % 32) // 2
    m = np.zeros((D, D), np.float32)
    m[s, f] = 1.0
    return m


_UNPERM = _build_unperm()


def kernel(x, efeat, degs, norm, params, edge_index):
    pad = E_PAD - E
    sd_p = jnp.stack([
        jnp.concatenate(
            [edge_index[i], jnp.zeros((pad,), jnp.int32)]).reshape(
                NW, NSUP, CPS, CH)
        for i in range(2)], axis=2)
    ne = jnp.concatenate([efeat, norm], axis=1)
    ne_p = jnp.concatenate(
        [ne, jnp.zeros((pad, 8), jnp.float32)], axis=0).reshape(
            NW, NSUP, SUP_E // 2, 16)
    zeros_t = jnp.zeros((ROWS_PT, D), jnp.bfloat16)
    unperm = jnp.asarray(_UNPERM)
    perm_t = jnp.asarray(_UNPERM.T)

    def layer(h, p):
        bias = jnp.zeros((8, D), jnp.float32)
        bias = bias.at[0].set(p['b']).at[1].set(p['be'])
        bias = bias.at[2].set(p['br']).at[3].set(p['root'][0])
        table, selfterm, res = _pre_call(h, p['W'], p['Wr'], bias, degs, perm_t)
        wmat = jnp.zeros((8, D), jnp.float32).at[:ED].set(p['We'])
        ft2 = _mp_call(table, sd_p, ne_p, wmat, zeros_t)
        gb = jnp.zeros((8, D), jnp.float32)
        gb = gb.at[0].set(p['gamma']).at[1].set(p['beta'])
        return _post_call(ft2, selfterm, res, gb, unperm)

    h = layer(x, params['layer0'])
    h = layer(h, params['layer1'])
    return h


# speedup vs baseline: 1.2089x; 1.2089x over previous
import functools

import jax
import jax.numpy as jnp
import numpy as np
from jax import lax
from jax.experimental import pallas as pl
from jax.experimental.pallas import tpu as pltpu
from jax.experimental.pallas import tpu_sc as plsc

N = 10000
D = 128
E = 320000
ED = 7

NC = 2
NS = 16
NW = NC * NS
CH = 128
CPS = 8
SUP_E = CPS * CH
NSUP = 10
EPT = NSUP * SUP_E
E_PAD = EPT * NW
ROWS_PT = 624
TAIL = N - NS * ROWS_PT
PHASES = 2
NCH = D // 16


_RB = 1000


def _pre_body(x_ref, w_ref, wr_ref, bias_ref, degs_ref, perm_ref,
              table_ref, self_ref, res_ref):
    x = x_ref[...]
    nf = jnp.dot(x, w_ref[...], preferred_element_type=jnp.float32) + bias_ref[0]
    table_ref[...] = jnp.dot(nf + bias_ref[1], perm_ref[...],
                             preferred_element_type=jnp.float32).astype(
                                 jnp.bfloat16)
    self_ref[...] = jnp.maximum(nf + bias_ref[3], 0.0) / degs_ref[...]
    res_ref[...] = jnp.maximum(
        jnp.dot(x, wr_ref[...], preferred_element_type=jnp.float32) + bias_ref[2],
        0.0)


def _pre_call(x, W, Wr, bias, degs, perm):
    return pl.pallas_call(
        _pre_body,
        grid=(N // _RB,),
        in_specs=[
            pl.BlockSpec((_RB, D), lambda i: (i, 0)),
            pl.BlockSpec((D, D), lambda i: (0, 0)),
            pl.BlockSpec((D, D), lambda i: (0, 0)),
            pl.BlockSpec((8, D), lambda i: (0, 0)),
            pl.BlockSpec((_RB, 1), lambda i: (i, 0)),
            pl.BlockSpec((D, D), lambda i: (0, 0)),
        ],
        out_specs=[
            pl.BlockSpec((_RB, D), lambda i: (i, 0)),
            pl.BlockSpec((_RB, D), lambda i: (i, 0)),
            pl.BlockSpec((_RB, D), lambda i: (i, 0)),
        ],
        out_shape=[
            jax.ShapeDtypeStruct((N, D), jnp.bfloat16),
            jax.ShapeDtypeStruct((N, D), jnp.float32),
            jax.ShapeDtypeStruct((N, D), jnp.float32),
        ],
    )(x, W, Wr, bias, degs, perm)


def _mp_body(table_h, sd_h, ne_h, w_h, z_h, out_h,
             w_v, sd_v, ne_v, g0, g1, s0, s1, acc0, acc1,
             gsem0, gsem1, ssem0, ssem1):
    cid = lax.axis_index("c")
    sid = lax.axis_index("s")
    wid = cid * NS + sid

    pltpu.sync_copy(w_h, w_v)

    def zero_acc():
        for acc in (acc0, acc1):
            pltpu.sync_copy(z_h.at[pl.ds(0, ROWS_PT)],
                            acc.at[pl.ds(sid * ROWS_PT, ROWS_PT)])

        @pl.when(sid == NS - 1)
        def _zero_tail():
            for acc in (acc0, acc1):
                pltpu.sync_copy(z_h.at[pl.ds(0, TAIL)],
                                acc.at[pl.ds(NS * ROWS_PT, TAIL)])

    wv = [[w_v[k, pl.ds(c * 16, 16)] for c in range(NCH)] for k in range(ED)]

    gbufs = (g0, g1)
    sbufs = (s0, s1)
    gsems = (gsem0, gsem1)
    ssems = (ssem0, ssem1)
    accs = (acc0, acc1)

    def superchunk(s, carry):
        pltpu.sync_copy(sd_h.at[wid, s], sd_v)
        pltpu.sync_copy(ne_h.at[wid, s], ne_v)
        pltpu.async_copy(table_h.at[sd_v.at[0, 0]], g0, gsem0)
        for j in range(CPS):
            b = j % 2
            gbuf = gbufs[b]
            sbuf = sbufs[b]
            acc = accs[b]
            pltpu.make_async_copy(
                table_h.at[sd_v.at[0, j]], gbuf, gsems[b]).wait()
            if j + 1 < CPS:
                pltpu.async_copy(table_h.at[sd_v.at[0, j + 1]],
                                 gbufs[1 - b], gsems[1 - b])
            if j >= 2:
                pltpu.make_async_copy(
                    sbuf, acc.at[sd_v.at[1, j - 2]], ssems[b]).wait()

            def pair_body(i2, _, j=j, gbuf=gbuf, sbuf=sbuf):
                nev = ne_v[j * (CH // 2) + i2, :]
                for half in range(2):
                    i = 2 * i2 + half
                    f = [nev[8 * half + k] for k in range(ED)]
                    nrm = nev[8 * half + ED]
                    vs = []
                    for c2 in range(NCH // 2):
                        ab = gbuf[i, pl.ds(c2 * 32, 32)]
                        va, vb = plsc.unpack(
                            ab, format=plsc.PackFormat.INTERLEAVED)
                        vs.append(va)
                        vs.append(vb)
                    out = []
                    for c in range(NCH):
                        v = vs[c]
                        for k in range(ED):
                            v = v + f[k] * wv[k][c]
                        out.append(jnp.maximum(v, 0.0) * nrm)
                    for c2 in range(NCH // 2):
                        packed = plsc.pack(out[2 * c2], out[2 * c2 + 1],
                                           format=plsc.PackFormat.INTERLEAVED)
                        sbuf[i, pl.ds(c2 * 32, 32)] = packed
                return 0

            lax.fori_loop(0, CH // 2, pair_body, 0)
            pltpu.async_copy(sbuf, acc.at[sd_v.at[1, j]], ssems[b], add=True)
        pltpu.make_async_copy(s0, accs[0].at[sd_v.at[1, CPS - 2]], ssem0).wait()
        pltpu.make_async_copy(s1, accs[1].at[sd_v.at[1, CPS - 1]], ssem1).wait()
        return carry

    for ph in range(PHASES):
        zero_acc()
        plsc.subcore_barrier()
        lax.fori_loop(ph * (NSUP // PHASES), (ph + 1) * (NSUP // PHASES),
                      superchunk, 0)
        plsc.subcore_barrier()
        for a in range(2):
            o = out_h.at[4 * ph + 2 * cid + a]
            pltpu.sync_copy(accs[a].at[pl.ds(sid * ROWS_PT, ROWS_PT)],
                            o.at[pl.ds(sid * ROWS_PT, ROWS_PT)])

        @pl.when(sid == NS - 1)
        def _copy_tail(ph=ph):
            for a in range(2):
                o = out_h.at[4 * ph + 2 * cid + a]
                pltpu.sync_copy(accs[a].at[pl.ds(NS * ROWS_PT, TAIL)],
                                o.at[pl.ds(NS * ROWS_PT, TAIL)])
        plsc.subcore_barrier()


_mp_call = pl.kernel(
    _mp_body,
    out_type=jax.ShapeDtypeStruct((2 * NC * PHASES, N, D), jnp.bfloat16),
    mesh=plsc.VectorSubcoreMesh(core_axis_name="c", subcore_axis_name="s"),
    compiler_params=pltpu.CompilerParams(use_tc_tiling_on_sc=False,
                                         needs_layout_passes=False),
    scratch_types=[
        pltpu.VMEM((8, D), jnp.float32),
        pltpu.VMEM((2, CPS, CH), jnp.int32),
        pltpu.VMEM((SUP_E // 2, 16), jnp.float32),
        pltpu.VMEM((CH, D), jnp.bfloat16),
        pltpu.VMEM((CH, D), jnp.bfloat16),
        pltpu.VMEM((CH, D), jnp.bfloat16),
        pltpu.VMEM((CH, D), jnp.bfloat16),
        pltpu.VMEM_SHARED((N, D), jnp.bfloat16),
        pltpu.VMEM_SHARED((N, D), jnp.bfloat16),
        pltpu.SemaphoreType.DMA,
        pltpu.SemaphoreType.DMA,
        pltpu.SemaphoreType.DMA,
        pltpu.SemaphoreType.DMA,
    ],
)


def _post_body(ft2_ref, self_ref, res_ref, gb_ref, unperm_ref, y_ref):
    fts = jnp.sum(ft2_ref[...].astype(jnp.float32), axis=0)
    ft = jnp.dot(fts, unperm_ref[...], preferred_element_type=jnp.float32)
    y = jnp.maximum(ft + self_ref[...], 0.0) + res_ref[...]
    mean = jnp.mean(y, axis=0, keepdims=True)
    var = jnp.mean((y - mean) * (y - mean), axis=0, keepdims=True)
    y_ref[...] = (y - mean) / jnp.sqrt(var + 1e-5) * gb_ref[0] + gb_ref[1]


def _post_call(ft2, selfterm, res, gb, unperm):
    return pl.pallas_call(
        _post_body,
        out_shape=jax.ShapeDtypeStruct((N, D), jnp.float32),
    )(ft2, selfterm, res, gb, unperm)


def _build_unperm():
    s = np.arange(D)
    f = 32 * (s // 32) + 16 * (s % 2) + (s % 32) // 2
    m = np.zeros((D, D), np.float32)
    m[s, f] = 1.0
    return m


_UNPERM = _build_unperm()


def kernel(x, efeat, degs, norm, params, edge_index):
    pad = E_PAD - E
    sd_p = jnp.stack([
        jnp.concatenate(
            [edge_index[i], jnp.zeros((pad,), jnp.int32)]).reshape(
                NW, NSUP, CPS, CH)
        for i in range(2)], axis=2)
    ne = jnp.concatenate([efeat, norm], axis=1)
    ne_p = jnp.concatenate(
        [ne, jnp.zeros((pad, 8), jnp.float32)], axis=0).reshape(
            NW, NSUP, SUP_E // 2, 16)
    zeros_t = jnp.zeros((ROWS_PT, D), jnp.bfloat16)
    unperm = jnp.asarray(_UNPERM)
    perm_t = jnp.asarray(_UNPERM.T)

    def layer(h, p):
        bias = jnp.zeros((8, D), jnp.float32)
        bias = bias.at[0].set(p['b']).at[1].set(p['be'])
        bias = bias.at[2].set(p['br']).at[3].set(p['root'][0])
        table, selfterm, res = _pre_call(h, p['W'], p['Wr'], bias, degs, perm_t)
        wmat = jnp.zeros((8, D), jnp.float32).at[:ED].set(p['We'])
        ft2 = _mp_call(table, sd_p, ne_p, wmat, zeros_t)
        gb = jnp.zeros((8, D), jnp.float32)
        gb = gb.at[0].set(p['gamma']).at[1].set(p['beta'])
        return _post_call(ft2, selfterm, res, gb, unperm)

    h = layer(x, params['layer0'])
    h = layer(h, params['layer1'])
    return h
